# baseline (speedup 1.0000x reference)
import jax
import jax.numpy as jnp
from jax import lax
from jax.experimental import pallas as pl
from jax.experimental.pallas import tpu as pltpu
from jax.experimental.pallas import tpu_sc as plsc


_B, _C, _HW = 8, 256, 4096
_Q = 1024
_NCH = 8
_CHT = 128
_LCACHE = 8
_MCACHE = 16
_KEEP = 48
_NEG_INIT = -1.0e9
_SENT = -2.0e9
_TAKEN = -3.0e9


def _iota16():
    return lax.broadcasted_iota(jnp.int32, (16,), 0)


def _splat_f(x):
    return jnp.full((16,), x, jnp.float32)


def _splat_i(x):
    return jnp.full((16,), x, jnp.int32)


def _shuffle(x, idx):
    dnums = lax.GatherDimensionNumbers(
        offset_dims=(), collapsed_slice_dims=(0,), start_index_map=(0,))
    return lax.gather(x, idx[:, None], dnums, (1,),
                      mode=lax.GatherScatterMode.PROMISE_IN_BOUNDS)


def _allmax(x):
    for s in (1, 2, 4, 8):
        x = jnp.maximum(x, _shuffle(x, jnp.bitwise_xor(_iota16(), s)))
    return x


def _allmin(x):
    for s in (1, 2, 4, 8):
        x = jnp.minimum(x, _shuffle(x, jnp.bitwise_xor(_iota16(), s)))
    return x


def _cache_insert(cache_s, cache_i, x, xi):
    cache_s, cache_i = list(cache_s), list(cache_i)
    for lvl in range(len(cache_s)):
        m = x > cache_s[lvl]
        ns = jnp.where(m, x, cache_s[lvl])
        x = jnp.where(m, cache_s[lvl], x)
        ni = jnp.where(m, xi, cache_i[lvl])
        xi = jnp.where(m, cache_i[lvl], xi)
        cache_s[lvl], cache_i[lvl] = ns, ni
    return cache_s, cache_i


def _refined_sqrt(x):
    xi = lax.bitcast_convert_type(x, jnp.int32)
    r = lax.bitcast_convert_type(
        _splat_i(0x5F3759DF) - lax.shift_right_logical(xi, 1), jnp.float32)
    for _ in range(2):
        r = r * (1.5 - 0.5 * x * r * r)
    y = x * r
    y = 0.5 * (y + x / y)
    c = y * 4097.0
    yh = c - (c - y)
    yl = y - yh
    p = y * y
    e = ((yh * yh - p) + 2.0 * yh * yl) + yl * yl
    d = (x - p) - e
    s = y + d / (y + y)
    return jnp.where(x > 0.0, s, 0.0)


def _sc_body(feat2d, feat1, mask5, out_hbm, fidx_hbm, cand_hbm,
             buf_a, buf_b, maskv, candv, mcand,
             fidxv, gidx, gbuf, tvecv,
             sem_a, sem_b, sem_g):
    core = lax.axis_index("c")
    sub = lax.axis_index("s")
    b = core * 4 + sub // 4
    q = sub % 4
    iota = _iota16()

    pltpu.sync_copy(mask5.at[b, q], maskv)

    def chunk_copy(buf, sem, cc):
        col0 = pl.multiple_of((q * _Q + cc * _CHT), _CHT)
        return pltpu.make_async_copy(
            feat2d.at[pl.ds(b * _C, _C), pl.ds(col0, _CHT)],
            buf, sem)

    def chunk_compute(buf, cc, cache_s, cache_i):
        def row_step(rb, accs):
            out = []
            for g in range(8):
                sq = []
                for r in range(16):
                    v = buf[rb * 16 + r, pl.ds(g * 16, 16)]
                    sq.append(v * v)
                while len(sq) > 1:
                    sq = [sq[i] + sq[i + 1] for i in range(0, len(sq), 2)]
                out.append(accs[g] + sq[0])
            return tuple(out)

        accs = lax.fori_loop(0, 16, row_step,
                             tuple(_splat_f(0.0) for _ in range(8)))
        for g in range(8):
            mv = maskv[cc, g, :]
            idxv = q * _Q + cc * _CHT + g * 16 + iota
            key = jnp.where(mv >= 0.5, accs[g],
                            -1.0 - idxv.astype(jnp.float32))
            cache_s, cache_i = _cache_insert(cache_s, cache_i, key, idxv)
        return cache_s, cache_i

    chunk_copy(buf_a, sem_a, 0).start()

    def super_step(it, carry):
        cache_s = list(carry[:_LCACHE])
        cache_i = list(carry[_LCACHE:])
        cc0 = it * 2
        cc1 = cc0 + 1
        chunk_copy(buf_b, sem_b, cc1).start()
        chunk_copy(buf_a, sem_a, cc0).wait()
        cache_s, cache_i = chunk_compute(buf_a, cc0, cache_s, cache_i)

        @pl.when(it < 3)
        def _():
            chunk_copy(buf_a, sem_a, cc0 + 2).start()

        chunk_copy(buf_b, sem_b, cc1).wait()
        cache_s, cache_i = chunk_compute(buf_b, cc1, cache_s, cache_i)
        return tuple(cache_s) + tuple(cache_i)

    init = (tuple(_splat_f(_NEG_INIT) for _ in range(_LCACHE))
            + tuple(_splat_i(0) for _ in range(_LCACHE)))
    fin = lax.fori_loop(0, 4, super_step, init)
    for lvl in range(_LCACHE):
        candv[lvl, :] = fin[lvl]
        candv[_LCACHE + lvl, :] = lax.bitcast_convert_type(
            fin[_LCACHE + lvl], jnp.float32)
    pltpu.sync_copy(candv, cand_hbm.at[core * 16 + sub])
    plsc.subcore_barrier()

    @pl.when(sub % 4 == 0)
    def _merge():
        b_loc = sub // 4
        for rr in range(4):
            pltpu.sync_copy(cand_hbm.at[core * 16 + b_loc * 4 + rr],
                            mcand.at[pl.ds(rr * 2 * _LCACHE, 2 * _LCACHE)])

        def ins_step(cchunk, carry):
            cs = list(carry[:_MCACHE])
            ci = list(carry[_MCACHE:])
            base = (cchunk // _LCACHE) * 2 * _LCACHE + cchunk % _LCACHE
            s = mcand[base, :]
            i = lax.bitcast_convert_type(mcand[base + _LCACHE, :], jnp.int32)
            cs, ci = _cache_insert(cs, ci, s, i)
            return tuple(cs) + tuple(ci)

        minit = (tuple(_splat_f(_SENT) for _ in range(_MCACHE))
                 + tuple(_splat_i(0) for _ in range(_MCACHE)))
        mfin = lax.fori_loop(0, 4 * _LCACHE, ins_step, minit)

        nkv = _KEEP // 16

        def ext_step(j, carry):
            cs = list(carry[:_MCACHE])
            ci = list(carry[_MCACHE:2 * _MCACHE])
            keep_s = list(carry[2 * _MCACHE:2 * _MCACHE + nkv])
            keep_i = list(carry[2 * _MCACHE + nkv:])
            m_b = _allmax(cs[0])
            win_b = _allmin(jnp.where(cs[0] == m_b, ci[0], _splat_i(_HW)))
            jvec = _splat_i(0) + j
            for v in range(nkv):
                sel = (iota + 16 * v) == jvec
                keep_s[v] = jnp.where(sel, m_b, keep_s[v])
                keep_i[v] = jnp.where(sel, win_b, keep_i[v])
            pop = ci[0] == win_b
            for lvl in range(_MCACHE - 1):
                cs[lvl] = jnp.where(pop, cs[lvl + 1], cs[lvl])
                ci[lvl] = jnp.where(pop, ci[lvl + 1], ci[lvl])
            cs[_MCACHE - 1] = jnp.where(pop, _splat_f(_SENT),
                                        cs[_MCACHE - 1])
            return tuple(cs) + tuple(ci) + tuple(keep_s) + tuple(keep_i)

        efin = lax.fori_loop(
            0, _KEEP, ext_step,
            mfin + tuple(_splat_f(_TAKEN) for _ in range(nkv))
            + tuple(_splat_i(0) for _ in range(nkv)))
        ks = list(efin[2 * _MCACHE:2 * _MCACHE + nkv])
        ki = list(efin[2 * _MCACHE + nkv:])
        ks = [jnp.where(s >= 0.0, _refined_sqrt(s), s) for s in ks]

        def rank_step(j, carry):
            ss = list(carry[:nkv])
            si = list(carry[nkv:2 * nkv])
            fid = list(carry[2 * nkv:])
            m = ss[0]
            for v in range(1, nkv):
                m = jnp.maximum(m, ss[v])
            m_b = _allmax(m)
            win = _splat_i(_HW)
            for v in range(nkv):
                win = jnp.minimum(win, jnp.where(ss[v] == m_b, si[v],
                                                 _splat_i(_HW)))
            win_b = _allmin(win)
            jvec = _splat_i(0) + j
            for v in range(2):
                sel = (iota + 16 * v) == jvec
                fid[v] = jnp.where(sel, win_b, fid[v])
            ss = [jnp.where(si[v] == win_b, _splat_f(_TAKEN), ss[v])
                  for v in range(nkv)]
            return tuple(ss) + tuple(si) + tuple(fid)

        rfin = lax.fori_loop(0, 32, rank_step,
                             tuple(ks) + tuple(ki)
                             + tuple(_splat_i(0) for _ in range(2)))
        fidxv[0, :] = rfin[2 * nkv]
        fidxv[1, :] = rfin[2 * nkv + 1]
        pltpu.sync_copy(fidxv, fidx_hbm.at[b])

    plsc.subcore_barrier()

    j0 = q * 8
    pltpu.sync_copy(fidx_hbm.at[b, q // 2, pl.ds((q % 2) * 8, 8)],
                    tvecv.at[pl.ds(0, 8)])
    tvec = tvecv[...]

    def build_step(tj, _):
        t_b = _shuffle(tvec, _splat_i(0) + tj)
        for half in range(2):
            for cl in range(8):
                ch = half * 128 + cl * 16 + iota
                evec = b * (_C * _HW) + ch * _HW + t_b
                gidx[tj * 2 + half, pl.ds(cl * 16, 16)] = evec
        return 0

    lax.fori_loop(0, 8, build_step, 0)

    def g_copy(ci16):
        return pltpu.make_async_copy(feat1.at[gidx.at[ci16]],
                                     gbuf.at[ci16], sem_g)

    def fire_step(ci16, _):
        g_copy(ci16).start()
        return 0

    lax.fori_loop(0, 16, fire_step, 0)

    def drain_step(ci16, _):
        g_copy(ci16).wait()
        return 0

    lax.fori_loop(0, 16, drain_step, 0)
    pltpu.sync_copy(gbuf, out_hbm.at[pl.ds((b * 32 + j0) * 2, 16)])


def kernel(feat, mask_rs, k):
    b, c, h, w = feat.shape
    hw = h * w
    feat2d = feat.reshape(b * c, hw)
    feat1 = feat.reshape(b * c * hw)
    mask_flat = mask_rs.reshape(b, hw)
    valid = jnp.sum(mask_flat, axis=1, keepdims=True) > 0.0
    mask_eff = jnp.where(valid, mask_flat, jnp.ones_like(mask_flat))
    mask5 = mask_eff.reshape(b, 4, _NCH, 8, 16)

    f = pl.kernel(
        _sc_body,
        out_type=(jax.ShapeDtypeStruct((b * 32 * 2, c // 2), jnp.float32),
                  jax.ShapeDtypeStruct((b, 2, 16), jnp.int32),
                  jax.ShapeDtypeStruct((32, 2 * _LCACHE, 16), jnp.float32)),
        mesh=plsc.VectorSubcoreMesh(core_axis_name="c", subcore_axis_name="s"),
        scratch_types=[
            pltpu.VMEM((_C, _CHT), jnp.float32),
            pltpu.VMEM((_C, _CHT), jnp.float32),
            pltpu.VMEM((_NCH, 8, 16), jnp.float32),
            pltpu.VMEM((2 * _LCACHE, 16), jnp.float32),
            pltpu.VMEM((8 * _LCACHE, 16), jnp.float32),
            pltpu.VMEM((2, 16), jnp.int32),
            pltpu.VMEM((16, 128), jnp.int32),
            pltpu.VMEM((16, 128), jnp.float32),
            pltpu.VMEM((16,), jnp.int32),
            pltpu.SemaphoreType.DMA,
            pltpu.SemaphoreType.DMA,
            pltpu.SemaphoreType.DMA,
        ],
    )
    tok, _, _ = f(feat2d, feat1, mask5)
    tok = tok.reshape(b, 32, c)
    return tok + jnp.asarray(k - 32, tok.dtype)


# speedup vs baseline: 1.0014x; 1.0014x over previous
import jax
import jax.numpy as jnp
from jax import lax
from jax.experimental import pallas as pl
from jax.experimental.pallas import tpu as pltpu
from jax.experimental.pallas import tpu_sc as plsc


_B, _C, _HW = 8, 256, 4096
_Q = 1024
_NCH = 8
_CHT = 128
_LCACHE = 8
_MCACHE = 16
_KEEP = 48
_NEG_INIT = -1.0e9
_SENT = -2.0e9
_TAKEN = -3.0e9


def _iota16():
    return lax.broadcasted_iota(jnp.int32, (16,), 0)


def _splat_f(x):
    return jnp.full((16,), x, jnp.float32)


def _splat_i(x):
    return jnp.full((16,), x, jnp.int32)


def _shuffle(x, idx):
    dnums = lax.GatherDimensionNumbers(
        offset_dims=(), collapsed_slice_dims=(0,), start_index_map=(0,))
    return lax.gather(x, idx[:, None], dnums, (1,),
                      mode=lax.GatherScatterMode.PROMISE_IN_BOUNDS)


def _allmax(x):
    for s in (1, 2, 4, 8):
        x = jnp.maximum(x, _shuffle(x, jnp.bitwise_xor(_iota16(), s)))
    return x


def _allmin(x):
    for s in (1, 2, 4, 8):
        x = jnp.minimum(x, _shuffle(x, jnp.bitwise_xor(_iota16(), s)))
    return x


def _cache_insert(cache_s, cache_i, x, xi):
    cache_s, cache_i = list(cache_s), list(cache_i)
    for lvl in range(len(cache_s)):
        m = x > cache_s[lvl]
        ns = jnp.where(m, x, cache_s[lvl])
        x = jnp.where(m, cache_s[lvl], x)
        ni = jnp.where(m, xi, cache_i[lvl])
        xi = jnp.where(m, cache_i[lvl], xi)
        cache_s[lvl], cache_i[lvl] = ns, ni
    return cache_s, cache_i


def _refined_sqrt(x):
    xi = lax.bitcast_convert_type(x, jnp.int32)
    r = lax.bitcast_convert_type(
        _splat_i(0x5F3759DF) - lax.shift_right_logical(xi, 1), jnp.float32)
    for _ in range(2):
        r = r * (1.5 - 0.5 * x * r * r)
    y = x * r
    y = 0.5 * (y + x / y)
    c = y * 4097.0
    yh = c - (c - y)
    yl = y - yh
    p = y * y
    e = ((yh * yh - p) + 2.0 * yh * yl) + yl * yl
    d = (x - p) - e
    s = y + d / (y + y)
    return jnp.where(x > 0.0, s, 0.0)


def _sc_body(feat2d, feat1, mask5, out_hbm, fidx_hbm, cand_hbm,
             buf_a, buf_b, maskv, candv, mcand,
             fidxv, gidx, gbuf, tvecv,
             sem_a, sem_b, sem_g):
    core = lax.axis_index("c")
    sub = lax.axis_index("s")
    b = core * 4 + sub // 4
    q = sub % 4
    iota = _iota16()

    pltpu.sync_copy(mask5.at[b, q], maskv)

    def chunk_copy(buf, sem, cc):
        col0 = pl.multiple_of((q * _Q + cc * _CHT), _CHT)
        return pltpu.make_async_copy(
            feat2d.at[pl.ds(b * _C, _C), pl.ds(col0, _CHT)],
            buf, sem)

    def chunk_compute(buf, cc, cache_s, cache_i):
        def row_step(rb, accs):
            out = []
            for g in range(8):
                sq = []
                for r in range(16):
                    v = buf[rb * 16 + r, pl.ds(g * 16, 16)]
                    sq.append(v * v)
                while len(sq) > 1:
                    sq = [sq[i] + sq[i + 1] for i in range(0, len(sq), 2)]
                out.append(accs[g] + sq[0])
            return tuple(out)

        accs = lax.fori_loop(0, 16, row_step,
                             tuple(_splat_f(0.0) for _ in range(8)))
        for g in range(8):
            mv = maskv[cc, g, :]
            idxv = q * _Q + cc * _CHT + g * 16 + iota
            key = jnp.where(mv >= 0.5, accs[g],
                            -1.0 - idxv.astype(jnp.float32))
            cache_s, cache_i = _cache_insert(cache_s, cache_i, key, idxv)
        return cache_s, cache_i

    chunk_copy(buf_a, sem_a, 0).start()

    def super_step(it, carry):
        cache_s = list(carry[:_LCACHE])
        cache_i = list(carry[_LCACHE:])
        cc0 = it * 2
        cc1 = cc0 + 1
        chunk_copy(buf_b, sem_b, cc1).start()
        chunk_copy(buf_a, sem_a, cc0).wait()
        cache_s, cache_i = chunk_compute(buf_a, cc0, cache_s, cache_i)

        @pl.when(it < 3)
        def _():
            chunk_copy(buf_a, sem_a, cc0 + 2).start()

        chunk_copy(buf_b, sem_b, cc1).wait()
        cache_s, cache_i = chunk_compute(buf_b, cc1, cache_s, cache_i)
        return tuple(cache_s) + tuple(cache_i)

    init = (tuple(_splat_f(_NEG_INIT) for _ in range(_LCACHE))
            + tuple(_splat_i(0) for _ in range(_LCACHE)))
    fin = lax.fori_loop(0, 4, super_step, init)
    for lvl in range(_LCACHE):
        candv[lvl, :] = fin[lvl]
        candv[_LCACHE + lvl, :] = lax.bitcast_convert_type(
            fin[_LCACHE + lvl], jnp.float32)
    pltpu.sync_copy(candv, cand_hbm.at[core * 16 + sub])
    plsc.subcore_barrier()

    @pl.when(sub % 4 == 0)
    def _merge():
        b_loc = sub // 4
        for rr in range(4):
            pltpu.sync_copy(cand_hbm.at[core * 16 + b_loc * 4 + rr],
                            mcand.at[pl.ds(rr * 2 * _LCACHE, 2 * _LCACHE)])

        def ins_step(cchunk, carry):
            cs = list(carry[:_MCACHE])
            ci = list(carry[_MCACHE:])
            base = (cchunk // _LCACHE) * 2 * _LCACHE + cchunk % _LCACHE
            s = mcand[base, :]
            i = lax.bitcast_convert_type(mcand[base + _LCACHE, :], jnp.int32)
            cs, ci = _cache_insert(cs, ci, s, i)
            return tuple(cs) + tuple(ci)

        minit = (tuple(_splat_f(_SENT) for _ in range(_MCACHE))
                 + tuple(_splat_i(0) for _ in range(_MCACHE)))
        mfin = lax.fori_loop(0, 4 * _LCACHE, ins_step, minit)

        nkv = _KEEP // 16

        def ext_step(j, carry):
            cs = list(carry[:_MCACHE])
            ci = list(carry[_MCACHE:2 * _MCACHE])
            keep_s = list(carry[2 * _MCACHE:2 * _MCACHE + nkv])
            keep_i = list(carry[2 * _MCACHE + nkv:])
            m_b = _allmax(cs[0])
            win_b = _allmin(jnp.where(cs[0] == m_b, ci[0], _splat_i(_HW)))
            jvec = _splat_i(0) + j
            for v in range(nkv):
                sel = (iota + 16 * v) == jvec
                keep_s[v] = jnp.where(sel, m_b, keep_s[v])
                keep_i[v] = jnp.where(sel, win_b, keep_i[v])
            pop = ci[0] == win_b
            for lvl in range(_MCACHE - 1):
                cs[lvl] = jnp.where(pop, cs[lvl + 1], cs[lvl])
                ci[lvl] = jnp.where(pop, ci[lvl + 1], ci[lvl])
            cs[_MCACHE - 1] = jnp.where(pop, _splat_f(_SENT),
                                        cs[_MCACHE - 1])
            return tuple(cs) + tuple(ci) + tuple(keep_s) + tuple(keep_i)

        efin = lax.fori_loop(
            0, _KEEP, ext_step,
            mfin + tuple(_splat_f(_TAKEN) for _ in range(nkv))
            + tuple(_splat_i(0) for _ in range(nkv)))
        ks = list(efin[2 * _MCACHE:2 * _MCACHE + nkv])
        ki = list(efin[2 * _MCACHE + nkv:])
        ks = [jnp.where(s >= 0.0, _refined_sqrt(s), s) for s in ks]

        def rank_step(j, carry):
            ss = list(carry[:nkv])
            si = list(carry[nkv:2 * nkv])
            fid = list(carry[2 * nkv:])
            m = ss[0]
            for v in range(1, nkv):
                m = jnp.maximum(m, ss[v])
            m_b = _allmax(m)
            win = _splat_i(_HW)
            for v in range(nkv):
                win = jnp.minimum(win, jnp.where(ss[v] == m_b, si[v],
                                                 _splat_i(_HW)))
            win_b = _allmin(win)
            jvec = _splat_i(0) + j
            for v in range(2):
                sel = (iota + 16 * v) == jvec
                fid[v] = jnp.where(sel, win_b, fid[v])
            ss = [jnp.where(si[v] == win_b, _splat_f(_TAKEN), ss[v])
                  for v in range(nkv)]
            return tuple(ss) + tuple(si) + tuple(fid)

        rfin = lax.fori_loop(0, 32, rank_step,
                             tuple(ks) + tuple(ki)
                             + tuple(_splat_i(0) for _ in range(2)))
        fidxv[0, :] = rfin[2 * nkv]
        fidxv[1, :] = rfin[2 * nkv + 1]
        pltpu.sync_copy(fidxv, fidx_hbm.at[b])

    plsc.subcore_barrier()

    j0 = q * 8
    pltpu.sync_copy(fidx_hbm.at[b, q // 2, pl.ds((q % 2) * 8, 8)],
                    tvecv.at[pl.ds(0, 8)])
    tvec = tvecv[...]

    def build_step(tj, _):
        t_b = _shuffle(tvec, _splat_i(0) + tj)
        for half in range(2):
            for cl in range(8):
                ch = half * 128 + cl * 16 + iota
                evec = b * (_C * _HW) + ch * _HW + t_b
                gidx[tj * 2 + half, pl.ds(cl * 16, 16)] = evec
        return 0

    lax.fori_loop(0, 8, build_step, 0)

    def g_copy(ci16):
        return pltpu.make_async_copy(feat1.at[gidx.at[ci16]],
                                     gbuf.at[ci16], sem_g)

    def fire_step(ci16, _):
        g_copy(ci16).start()
        return 0

    lax.fori_loop(0, 16, fire_step, 0)

    def drain_step(ci16, _):
        g_copy(ci16).wait()
        return 0

    lax.fori_loop(0, 16, drain_step, 0)
    pltpu.sync_copy(gbuf, out_hbm.at[pl.ds((b * 32 + j0) * 2, 16)])


def kernel(feat, mask_rs, k):
    b, c, h, w = feat.shape
    hw = h * w
    feat2d = feat.reshape(b * c, hw)
    feat1 = feat.reshape(b * c * hw)
    mask_flat = mask_rs.reshape(b, hw)
    valid = jnp.sum(mask_flat, axis=1, keepdims=True) > 0.0
    mask_eff = jnp.where(valid, mask_flat, jnp.ones_like(mask_flat))
    mask5 = mask_eff.reshape(b, 4, _NCH, 8, 16)

    f = pl.kernel(
        _sc_body,
        out_type=(jax.ShapeDtypeStruct((b * 32 * 2, c // 2), jnp.float32),
                  jax.ShapeDtypeStruct((b, 2, 16), jnp.int32),
                  jax.ShapeDtypeStruct((32, 2 * _LCACHE, 16), jnp.float32)),
        mesh=plsc.VectorSubcoreMesh(core_axis_name="c", subcore_axis_name="s"),
        compiler_params=pltpu.CompilerParams(use_tc_tiling_on_sc=True),
        scratch_types=[
            pltpu.VMEM((_C, _CHT), jnp.float32),
            pltpu.VMEM((_C, _CHT), jnp.float32),
            pltpu.VMEM((_NCH, 8, 16), jnp.float32),
            pltpu.VMEM((2 * _LCACHE, 16), jnp.float32),
            pltpu.VMEM((8 * _LCACHE, 16), jnp.float32),
            pltpu.VMEM((2, 16), jnp.int32),
            pltpu.VMEM((16, 128), jnp.int32),
            pltpu.VMEM((16, 128), jnp.float32),
            pltpu.VMEM((16,), jnp.int32),
            pltpu.SemaphoreType.DMA,
            pltpu.SemaphoreType.DMA,
            pltpu.SemaphoreType.DMA,
        ],
    )
    tok, _, _ = f(feat2d, feat1, mask5)
    tok = tok.reshape(b, 32, c)
    return tok + jnp.asarray(k - 32, tok.dtype)
